# Initial kernel scaffold; baseline (speedup 1.0000x reference)
import jax
import jax.numpy as jnp
from jax.experimental import pallas as pl


def kernel(x, edge_index, batch, c0_W1, c0_b1, c0_g, c0_beta, c0_W2, c0_b2, cs_W1, cs_b1, cs_g, cs_beta, cs_W2, cs_b2, f_W1, f_b1, f_W2, f_b2):
    raise NotImplementedError("write your pallas kernel here")


# speedup vs baseline: 3.4812x; 3.4812x over previous
import functools

import jax
import jax.numpy as jnp
from jax import lax
from jax.experimental import pallas as pl
from jax.experimental.pallas import tpu as pltpu
from jax.experimental.pallas import tpu_sc as plsc

N = 10000
E = 320000
G = 128
H = 64
BN_EPS = 1e-5

NC = 2
NS = 16
NW = NC * NS
CHUNK = 128
E_PAD = ((E + NW * CHUNK - 1) // (NW * CHUNK)) * (NW * CHUNK)
PER_W = E_PAD // NW
N_CHUNKS = PER_W // CHUNK
N_PAD = 10112
RPT = N_PAD // NS
D = 128


def _make_seg_sum(n_rows):
    mesh = plsc.VectorSubcoreMesh(core_axis_name="c", subcore_axis_name="s")

    @functools.partial(
        pl.kernel,
        mesh=mesh,
        out_type=jax.ShapeDtypeStruct((NC, N_PAD, D), jnp.float32),
        scratch_types=[
            pltpu.VMEM((CHUNK,), jnp.int32),
            pltpu.VMEM((CHUNK,), jnp.int32),
            pltpu.VMEM((CHUNK, D), jnp.float32),
            pltpu.VMEM_SHARED((N_PAD, D), jnp.float32),
            pltpu.SemaphoreType.DMA,
        ],
    )
    def seg_sum(h_hbm, src_hbm, dst_hbm, zeros_hbm, out_hbm,
                sidx, didx, rows, acc, sem):
        ci = lax.axis_index("c")
        si = lax.axis_index("s")
        wid = ci * NS + si
        r0 = si * RPT
        pltpu.sync_copy(zeros_hbm.at[pl.ds(r0, RPT)], acc.at[pl.ds(r0, RPT)])
        plsc.subcore_barrier()

        base = wid * PER_W

        def body(j, carry):
            off = base + j * CHUNK
            pltpu.sync_copy(src_hbm.at[pl.ds(off, CHUNK)], sidx)
            pltpu.sync_copy(dst_hbm.at[pl.ds(off, CHUNK)], didx)
            pltpu.async_copy(h_hbm.at[sidx], rows, sem).wait()
            pltpu.sync_copy(rows, acc.at[didx], add=True)
            return carry

        lax.fori_loop(0, N_CHUNKS, body, 0)
        plsc.subcore_barrier()
        pltpu.sync_copy(acc.at[pl.ds(r0, RPT)], out_hbm.at[ci, pl.ds(r0, RPT)])

    return seg_sum


def _mlp_bn(u, W1, b1, g, beta, W2, b2):
    h1 = jnp.dot(u, W1, preferred_element_type=jnp.float32) + b1
    mu = jnp.mean(h1, axis=0, keepdims=True)
    var = jnp.mean(jnp.square(h1 - mu), axis=0, keepdims=True)
    hn = (h1 - mu) * lax.rsqrt(var + BN_EPS) * g + beta
    return jnp.dot(jnp.maximum(hn, 0.0), W2, preferred_element_type=jnp.float32) + b2


def _layer0_body(x_ref, p_ref, W1_ref, b1_ref, g_ref, beta_ref, W2_ref, b2_ref,
                 out_ref):
    u = x_ref[...] + p_ref[0, :N, :] + p_ref[1, :N, :]
    h2 = _mlp_bn(u, W1_ref[...], b1_ref[...], g_ref[...], beta_ref[...],
                 W2_ref[...], b2_ref[...])
    out_ref[...] = jnp.zeros((N_PAD, D), jnp.float32)
    out_ref[:N, :H] = jnp.maximum(h2, 0.0)


def _layer_body(x_ref, p_ref, W1_ref, b1_ref, g_ref, beta_ref, W2_ref, b2_ref,
                out_ref):
    u = x_ref[:N, :H] + p_ref[0, :N, :H] + p_ref[1, :N, :H]
    h2 = _mlp_bn(u, W1_ref[...], b1_ref[...], g_ref[...], beta_ref[...],
                 W2_ref[...], b2_ref[...])
    out_ref[...] = jnp.zeros((N_PAD, D), jnp.float32)
    out_ref[:N, :H] = jnp.maximum(h2, 0.0)


def _final_body(x_ref, p_ref, W1_ref, b1_ref, g_ref, beta_ref, W2_ref, b2_ref,
                batch_ref, fW1_ref, fb1_ref, fW2_ref, fb2_ref, out_ref):
    u = x_ref[:N, :H] + p_ref[0, :N, :H] + p_ref[1, :N, :H]
    h = jnp.maximum(
        _mlp_bn(u, W1_ref[...], b1_ref[...], g_ref[...], beta_ref[...],
                W2_ref[...], b2_ref[...]), 0.0)
    gids = lax.broadcasted_iota(jnp.int32, (1, G), 1)
    onehot = (batch_ref[...] == gids).astype(jnp.float32)
    pooled = lax.dot_general(onehot, h, (((0,), (0,)), ((), ())),
                             preferred_element_type=jnp.float32)
    z = jnp.maximum(jnp.dot(pooled, fW1_ref[...],
                            preferred_element_type=jnp.float32) + fb1_ref[...], 0.0)
    out_ref[...] = jnp.dot(z, fW2_ref[...],
                           preferred_element_type=jnp.float32) + fb2_ref[...]


def _layer_tc(body, x, p, W1, b1, g, beta, W2, b2):
    return pl.pallas_call(
        body,
        out_shape=jax.ShapeDtypeStruct((N_PAD, D), jnp.float32),
    )(x, p, W1, b1.reshape(1, -1), g.reshape(1, -1), beta.reshape(1, -1),
      W2, b2.reshape(1, -1))


def _final_tc(x, p, W1, b1, g, beta, W2, b2, batch, fW1, fb1, fW2, fb2):
    return pl.pallas_call(
        _final_body,
        out_shape=jax.ShapeDtypeStruct((G, fW2.shape[1]), jnp.float32),
    )(x, p, W1, b1.reshape(1, -1), g.reshape(1, -1), beta.reshape(1, -1),
      W2, b2.reshape(1, -1), batch.reshape(-1, 1),
      fW1, fb1.reshape(1, -1), fW2, fb2.reshape(1, -1))


def kernel(x, edge_index, batch, c0_W1, c0_b1, c0_g, c0_beta, c0_W2, c0_b2,
           cs_W1, cs_b1, cs_g, cs_beta, cs_W2, cs_b2, f_W1, f_b1, f_W2, f_b2):
    pad = E_PAD - E
    src = jnp.concatenate([edge_index[0], jnp.zeros((pad,), jnp.int32)])
    dst = jnp.concatenate([edge_index[1], jnp.full((pad,), N, jnp.int32)])
    zeros = jnp.zeros((N_PAD, D), jnp.float32)

    seg_x = _make_seg_sum(N)
    seg_h = _make_seg_sum(N_PAD)

    p = seg_x(x, src, dst, zeros)
    h = _layer_tc(_layer0_body, x, p, c0_W1, c0_b1, c0_g, c0_beta, c0_W2, c0_b2)
    for i in range(3):
        p = seg_h(h, src, dst, zeros)
        h = _layer_tc(_layer_body, h, p, cs_W1[i], cs_b1[i], cs_g[i],
                      cs_beta[i], cs_W2[i], cs_b2[i])
    p = seg_h(h, src, dst, zeros)
    return _final_tc(h, p, cs_W1[3], cs_b1[3], cs_g[3], cs_beta[3],
                     cs_W2[3], cs_b2[3], batch, f_W1, f_b1, f_W2, f_b2)
